# Initial kernel scaffold; baseline (speedup 1.0000x reference)
import jax
import jax.numpy as jnp
from jax.experimental import pallas as pl


def kernel(x, si0, si1, si2, si3, dr0, dr1, dr2, dr3, dc0, dc1, dc2, dc3, dv0, dv1, dv2, dv3, W0, b0, W1, b1, W2, b2, W3, b3, Wlat, blat, Wcls, bcls):
    raise NotImplementedError("write your pallas kernel here")


# speedup vs baseline: 1.4733x; 1.4733x over previous
import functools

import jax
import jax.numpy as jnp
from jax.experimental import pallas as pl
from jax.experimental.pallas import tpu as pltpu

LEVELS = [10000, 2500, 625, 160, 40]
CH = [3, 32, 64, 128, 256]
SPIRAL = 9
BATCH = 64
LATENT = 256
NUM_OUT = 10

BLK_M = [2500, 625, 160, 40]


def _level_body(g_ref, w_ref, b_ref, dv_ref, out_ref):
    g = g_ref[0]
    y = jnp.dot(g, w_ref[...], preferred_element_type=jnp.float32)
    y = y + b_ref[...]
    y = jnp.where(y > 0, y, jnp.exp(y) - 1.0)
    y = y * dv_ref[...]
    blk_e, co = y.shape
    y = y.reshape(blk_e // 3, 3, co)
    out_ref[0] = jnp.sum(y, axis=1)


def _level_call(g, w, b, dv, m, blk_m):
    bsz, e, kdim = g.shape
    co = w.shape[1]
    grid = (bsz, m // blk_m)
    return pl.pallas_call(
        _level_body,
        grid=grid,
        in_specs=[
            pl.BlockSpec((1, 3 * blk_m, kdim), lambda ib, im: (ib, im, 0)),
            pl.BlockSpec((kdim, co), lambda ib, im: (0, 0)),
            pl.BlockSpec((1, co), lambda ib, im: (0, 0)),
            pl.BlockSpec((3 * blk_m, 1), lambda ib, im: (im, 0)),
        ],
        out_specs=pl.BlockSpec((1, blk_m, co), lambda ib, im: (ib, im, 0)),
        out_shape=jax.ShapeDtypeStruct((bsz, m, co), jnp.float32),
    )(g, w, b.reshape(1, co), dv.reshape(e, 1))


def _head_body(h_ref, wlat_ref, blat_ref, wcls_ref, bcls_ref, out_ref):
    h = h_ref[...]
    z = jnp.dot(h, wlat_ref[...], preferred_element_type=jnp.float32)
    z = z + blat_ref[...]
    out_ref[...] = jnp.dot(z, wcls_ref[...], preferred_element_type=jnp.float32) + bcls_ref[...]


def _head_call(h, wlat, blat, wcls, bcls):
    bsz, flat = h.shape
    return pl.pallas_call(
        _head_body,
        out_shape=jax.ShapeDtypeStruct((bsz, NUM_OUT), jnp.float32),
    )(h, wlat, blat.reshape(1, LATENT), wcls, bcls.reshape(1, NUM_OUT))


def kernel(x, si0, si1, si2, si3, dr0, dr1, dr2, dr3, dc0, dc1, dc2, dc3,
           dv0, dv1, dv2, dv3, W0, b0, W1, b1, W2, b2, W3, b3,
           Wlat, blat, Wcls, bcls):
    si = (si0, si1, si2, si3)
    dc = (dc0, dc1, dc2, dc3)
    dv = (dv0, dv1, dv2, dv3)
    Ws = (W0, W1, W2, W3)
    bs = (b0, b1, b2, b3)
    h = x
    for i in range(4):
        m = LEVELS[i + 1]
        c = CH[i]
        sic = jnp.take(si[i], dc[i], axis=0)
        g = jnp.take(h, sic.reshape(-1), axis=1)
        g = g.reshape(BATCH, 3 * m, SPIRAL * c)
        h = _level_call(g, Ws[i], bs[i], dv[i], m, BLK_M[i])
    hflat = h.reshape(BATCH, LEVELS[4] * CH[4])
    return _head_call(hflat, Wlat, blat, Wcls, bcls)
